# baseline (speedup 1.0000x reference)
import functools

import jax
import jax.numpy as jnp
from jax.experimental import pallas as pl


def _top2_weights(gates, iota):
    E = gates.shape[0]
    m1 = jnp.max(gates, axis=0, keepdims=True)
    i1 = jnp.min(jnp.where(gates == m1, iota, E), axis=0, keepdims=True)
    oh1 = iota == i1
    masked = jnp.where(oh1, -1.0, gates)
    m2 = jnp.max(masked, axis=0, keepdims=True)
    i2 = jnp.min(jnp.where(masked == m2, iota, E), axis=0, keepdims=True)
    oh2 = iota == i2
    return jnp.where(oh1, m1, 0.0) + jnp.where(oh2, m2, 0.0)


def _softmax0(lg):
    m = jnp.max(lg, axis=0, keepdims=True)
    e = jnp.exp(lg - m)
    return e / jnp.sum(e, axis=0, keepdims=True)


def _moe_kernel(x1_ref, x2_ref, n1_ref, n2_ref,
                w1_ref, b1_ref, ex1_ref, c1_ref, gw1_ref, gb1_ref,
                w2_ref, b2_ref, ex2_ref, c2_ref, gw2_ref, gb2_ref,
                cb_ref, out_ref):
    f32 = jnp.float32
    x1 = x1_ref[...]
    x2 = x2_ref[...]
    E = gw1_ref.shape[1]
    N = x1.shape[0]
    iota = jax.lax.broadcasted_iota(jnp.int32, (E, N), 0)

    t1 = jnp.maximum(jnp.dot(x1, w1_ref[...], preferred_element_type=f32)
                     + b1_ref[...], 0.0)
    lg1 = (jnp.dot(x1, gw1_ref[...], preferred_element_type=f32)
           + gb1_ref[...] + n1_ref[...])
    w1T = _top2_weights(_softmax0(lg1.T), iota)
    extra1 = jnp.sum(w1T * c1_ref[...], axis=0, keepdims=True)
    q1 = jnp.dot(w1T.T, ex1_ref[...], preferred_element_type=f32)
    acc1 = jnp.sum(q1 * t1, axis=1)[None, :] + extra1

    t2 = jnp.maximum(jnp.dot(x2, w2_ref[...], preferred_element_type=f32)
                     + b2_ref[...], 0.0)
    lg2 = (jnp.dot(x2, gw2_ref[...], preferred_element_type=f32)
           + gb2_ref[...] + n2_ref[...])
    w2T = _top2_weights(_softmax0(lg2.T), iota)
    extra2 = jnp.sum(w2T * c2_ref[...], axis=0, keepdims=True)
    q2 = jnp.dot(w2T.T, ex2_ref[...], preferred_element_type=f32)
    acc2 = jnp.sum(q2 * t2, axis=1)[None, :] + extra2

    out_ref[...] = acc1 + acc2 + cb_ref[...]


@functools.partial(jax.jit, static_argnames=("block_n",))
def _moe_fused(x1, x2, noise1, noise2,
               w1s, b1s, ex1, c1, gw1, gb1,
               w2s, b2s, ex2, c2, gw2, gb2, cb, block_n=4096):
    B, D1 = x1.shape
    D2 = x2.shape[1]
    E = gw1.shape[1]
    EHE = w1s.shape[1]
    grid = (B // block_n,)
    row = lambda i: (i, 0)
    col = lambda i: (0, i)
    fixed = lambda i: (0, 0)
    out = pl.pallas_call(
        _moe_kernel,
        grid=grid,
        in_specs=[
            pl.BlockSpec((block_n, D1), row),
            pl.BlockSpec((block_n, D2), row),
            pl.BlockSpec((block_n, E), row),
            pl.BlockSpec((block_n, E), row),
            pl.BlockSpec((D1, EHE), fixed),
            pl.BlockSpec((1, EHE), fixed),
            pl.BlockSpec((E, EHE), fixed),
            pl.BlockSpec((E, 1), fixed),
            pl.BlockSpec((D1, E), fixed),
            pl.BlockSpec((1, E), fixed),
            pl.BlockSpec((D2, EHE), fixed),
            pl.BlockSpec((1, EHE), fixed),
            pl.BlockSpec((E, EHE), fixed),
            pl.BlockSpec((E, 1), fixed),
            pl.BlockSpec((D2, E), fixed),
            pl.BlockSpec((1, E), fixed),
            pl.BlockSpec((1, 1), fixed),
        ],
        out_specs=pl.BlockSpec((1, block_n), col),
        out_shape=jax.ShapeDtypeStruct((1, B), jnp.float32),
    )(x1, x2, noise1, noise2,
      w1s, b1s, ex1, c1, gw1, gb1,
      w2s, b2s, ex2, c2, gw2, gb2, cb)
    return out.reshape(B, 1)


def kernel(x1, x2, noise1, noise2, E1_W1, E1_b1, E1_W2, E1_b2,
           E2_W1, E2_b1, E2_W2, E2_b2, G1_W, G1_b, G2_W, G2_b, C_W, C_b):
    E, D1, EH = E1_W1.shape
    ED = E1_W2.shape[2]
    D2 = E2_W1.shape[1]

    w1f = jnp.transpose(E1_W1, (1, 0, 2)).reshape(D1, E * EH)
    w2f = jnp.transpose(E2_W1, (1, 0, 2)).reshape(D2, E * EH)
    cw1 = C_W[:ED, 0]
    cw2 = C_W[ED:, 0]
    u1 = jnp.einsum('ehf,f->eh', E1_W2, cw1)
    u2 = jnp.einsum('ehf,f->eh', E2_W2, cw2)
    au1 = jnp.abs(u1).reshape(1, E * EH)
    au2 = jnp.abs(u2).reshape(1, E * EH)
    eye = jnp.eye(E, dtype=jnp.float32)
    ex1 = (eye[:, :, None] * jnp.sign(u1)[None, :, :]).reshape(E, E * EH)
    ex2 = (eye[:, :, None] * jnp.sign(u2)[None, :, :]).reshape(E, E * EH)
    w1s = w1f * au1
    w2s = w2f * au2
    b1s = E1_b1.reshape(1, E * EH) * au1
    b2s = E2_b1.reshape(1, E * EH) * au2
    c1 = (E1_b2 @ cw1).reshape(E, 1)
    c2 = (E2_b2 @ cw2).reshape(E, 1)
    gb1 = G1_b.reshape(1, E)
    gb2 = G2_b.reshape(1, E)
    cb = C_b.reshape(1, 1)

    return _moe_fused(x1, x2, noise1, noise2,
                      w1s, b1s, ex1, c1, G1_W, gb1,
                      w2s, b2s, ex2, c2, G2_W, gb2, cb)


# speedup vs baseline: 1.1485x; 1.1485x over previous
import functools

import jax
import jax.numpy as jnp
from jax.experimental import pallas as pl
from jax.experimental.pallas import tpu as pltpu


def _top2_mix(gates, s, iota):
    E = gates.shape[0]
    m1 = jnp.max(gates, axis=0, keepdims=True)
    i1 = jnp.min(jnp.where(gates == m1, iota, E), axis=0, keepdims=True)
    oh1 = iota == i1
    masked = jnp.where(oh1, -1.0, gates)
    m2 = jnp.max(masked, axis=0, keepdims=True)
    i2 = jnp.min(jnp.where(masked == m2, iota, E), axis=0, keepdims=True)
    oh2 = iota == i2
    w = jnp.where(oh1, m1, 0.0) + jnp.where(oh2, m2, 0.0)
    return jnp.sum(w * s, axis=0, keepdims=True)


def _softmax0(lg):
    m = jnp.max(lg, axis=0, keepdims=True)
    e = jnp.exp(lg - m)
    return e / jnp.sum(e, axis=0, keepdims=True)


def _expert_sums(t, E):
    C = t.shape[1]
    a = t
    width = C
    while width > 128:
        half = width // 2
        a = a[:, :half] + a[:, half:width]
        width = half
    while width > E:
        half = width // 2
        a = a + pltpu.roll(a, a.shape[1] - half, 1)
        width = half
    return a[:, :E]


def _moe_kernel(x1_ref, x2_ref, n1_ref, n2_ref,
                w1_ref, b1_ref, u1_ref, c1_ref, gw1_ref, gb1_ref,
                w2_ref, b2_ref, u2_ref, c2_ref, gw2_ref, gb2_ref,
                cb_ref, out_ref):
    f32 = jnp.float32
    x1 = x1_ref[...]
    x2 = x2_ref[...]
    E = gw1_ref.shape[1]
    N = x1.shape[0]
    iota = jax.lax.broadcasted_iota(jnp.int32, (E, N), 0)

    t1 = jnp.maximum(jnp.dot(x1, w1_ref[...], preferred_element_type=f32)
                     + b1_ref[...], 0.0) * u1_ref[...]
    s1 = _expert_sums(t1, E) + c1_ref[...]
    lg1 = (jnp.dot(x1, gw1_ref[...], preferred_element_type=f32)
           + gb1_ref[...] + n1_ref[...])
    g1 = _softmax0(lg1.T)
    acc1 = _top2_mix(g1, s1.T, iota)

    t2 = jnp.maximum(jnp.dot(x2, w2_ref[...], preferred_element_type=f32)
                     + b2_ref[...], 0.0) * u2_ref[...]
    s2 = _expert_sums(t2, E) + c2_ref[...]
    lg2 = (jnp.dot(x2, gw2_ref[...], preferred_element_type=f32)
           + gb2_ref[...] + n2_ref[...])
    g2 = _softmax0(lg2.T)
    acc2 = _top2_mix(g2, s2.T, iota)

    out_ref[...] = acc1 + acc2 + cb_ref[...]


@functools.partial(jax.jit, static_argnames=("block_n",))
def _moe_fused(x1, x2, noise1, noise2,
               w1p, b1p, u1p, c1, gw1, gb1,
               w2p, b2p, u2p, c2, gw2, gb2, cb, block_n=4096):
    B, D1 = x1.shape
    D2 = x2.shape[1]
    E = gw1.shape[1]
    EHE = w1p.shape[1]
    grid = (B // block_n,)
    row = lambda i: (i, 0)
    col = lambda i: (0, i)
    fixed = lambda i: (0, 0)
    out = pl.pallas_call(
        _moe_kernel,
        grid=grid,
        in_specs=[
            pl.BlockSpec((block_n, D1), row),
            pl.BlockSpec((block_n, D2), row),
            pl.BlockSpec((block_n, E), row),
            pl.BlockSpec((block_n, E), row),
            pl.BlockSpec((D1, EHE), fixed),
            pl.BlockSpec((1, EHE), fixed),
            pl.BlockSpec((1, EHE), fixed),
            pl.BlockSpec((1, E), fixed),
            pl.BlockSpec((D1, E), fixed),
            pl.BlockSpec((1, E), fixed),
            pl.BlockSpec((D2, EHE), fixed),
            pl.BlockSpec((1, EHE), fixed),
            pl.BlockSpec((1, EHE), fixed),
            pl.BlockSpec((1, E), fixed),
            pl.BlockSpec((D2, E), fixed),
            pl.BlockSpec((1, E), fixed),
            pl.BlockSpec((1, 1), fixed),
        ],
        out_specs=pl.BlockSpec((1, block_n), col),
        out_shape=jax.ShapeDtypeStruct((1, B), jnp.float32),
    )(x1, x2, noise1, noise2,
      w1p, b1p, u1p, c1, gw1, gb1,
      w2p, b2p, u2p, c2, gw2, gb2, cb)
    return out.reshape(B, 1)


def kernel(x1, x2, noise1, noise2, E1_W1, E1_b1, E1_W2, E1_b2,
           E2_W1, E2_b1, E2_W2, E2_b2, G1_W, G1_b, G2_W, G2_b, C_W, C_b):
    E, D1, EH = E1_W1.shape
    ED = E1_W2.shape[2]
    D2 = E2_W1.shape[1]

    w1p = jnp.transpose(E1_W1, (1, 2, 0)).reshape(D1, EH * E)
    w2p = jnp.transpose(E2_W1, (1, 2, 0)).reshape(D2, EH * E)
    b1p = jnp.transpose(E1_b1, (1, 0)).reshape(1, EH * E)
    b2p = jnp.transpose(E2_b1, (1, 0)).reshape(1, EH * E)
    cw1 = C_W[:ED, 0]
    cw2 = C_W[ED:, 0]
    u1p = jnp.einsum('ehf,f->he', E1_W2, cw1).reshape(1, EH * E)
    u2p = jnp.einsum('ehf,f->he', E2_W2, cw2).reshape(1, EH * E)
    c1 = (E1_b2 @ cw1).reshape(1, E)
    c2 = (E2_b2 @ cw2).reshape(1, E)
    gb1 = G1_b.reshape(1, E)
    gb2 = G2_b.reshape(1, E)
    cb = C_b.reshape(1, 1)

    return _moe_fused(x1, x2, noise1, noise2,
                      w1p, b1p, u1p, c1, G1_W, gb1,
                      w2p, b2p, u2p, c2, G2_W, gb2, cb)


# speedup vs baseline: 1.5140x; 1.3183x over previous
import functools

import jax
import jax.numpy as jnp
from jax.experimental import pallas as pl
from jax.experimental.pallas import tpu as pltpu


def _top2_mix(gates, s, iota):
    E = gates.shape[0]
    m1 = jnp.max(gates, axis=0, keepdims=True)
    i1 = jnp.min(jnp.where(gates == m1, iota, E), axis=0, keepdims=True)
    oh1 = iota == i1
    masked = jnp.where(oh1, -1.0, gates)
    m2 = jnp.max(masked, axis=0, keepdims=True)
    i2 = jnp.min(jnp.where(masked == m2, iota, E), axis=0, keepdims=True)
    oh2 = iota == i2
    w = jnp.where(oh1, m1, 0.0) + jnp.where(oh2, m2, 0.0)
    return jnp.sum(w * s, axis=0, keepdims=True)


def _softmax0(lg):
    m = jnp.max(lg, axis=0, keepdims=True)
    e = jnp.exp(lg - m)
    return e / jnp.sum(e, axis=0, keepdims=True)


def _moe_kernel(x1_ref, x2_ref, n1_ref, n2_ref,
                w1_ref, b1_ref, u1_ref, c1_ref, gb1_ref,
                w2_ref, b2_ref, u2_ref, c2_ref, gb2_ref,
                cb_ref, out_ref, h1_scr, h2_scr):
    f32 = jnp.float32
    x1 = x1_ref[...]
    x2 = x2_ref[...]
    E = c1_ref.shape[1]
    N = x1.shape[0]
    iota = jax.lax.broadcasted_iota(jnp.int32, (E, N), 0)

    EH = u1_ref.shape[0]

    y1 = jnp.dot(x1, w1_ref[...], preferred_element_type=f32)
    h1_scr[...] = jnp.maximum(y1[:, :EH] + b1_ref[...], 0.0)
    s1 = jnp.dot(h1_scr[...], u1_ref[...], preferred_element_type=f32) + c1_ref[...]
    lg1 = y1[:, EH:] + gb1_ref[...] + n1_ref[...]
    g1 = _softmax0(lg1.T)
    acc1 = _top2_mix(g1, s1.T, iota)

    y2 = jnp.dot(x2, w2_ref[...], preferred_element_type=f32)
    h2_scr[...] = jnp.maximum(y2[:, :EH] + b2_ref[...], 0.0)
    s2 = jnp.dot(h2_scr[...], u2_ref[...], preferred_element_type=f32) + c2_ref[...]
    lg2 = y2[:, EH:] + gb2_ref[...] + n2_ref[...]
    g2 = _softmax0(lg2.T)
    acc2 = _top2_mix(g2, s2.T, iota)

    out_ref[...] = acc1 + acc2 + cb_ref[...]


@functools.partial(jax.jit, static_argnames=("block_n",))
def _moe_fused(x1, x2, noise1, noise2,
               w1c, b1f, u1, c1, gb1,
               w2c, b2f, u2, c2, gb2, cb, block_n=4096):
    B, D1 = x1.shape
    D2 = x2.shape[1]
    E = c1.shape[1]
    EH = u1.shape[0]
    grid = (B // block_n,)
    row = lambda i: (i, 0)
    col = lambda i: (0, i)
    fixed = lambda i: (0, 0)
    out = pl.pallas_call(
        _moe_kernel,
        grid=grid,
        in_specs=[
            pl.BlockSpec((block_n, D1), row),
            pl.BlockSpec((block_n, D2), row),
            pl.BlockSpec((block_n, E), row),
            pl.BlockSpec((block_n, E), row),
            pl.BlockSpec((D1, EH + E), fixed),
            pl.BlockSpec((1, EH), fixed),
            pl.BlockSpec((EH, E), fixed),
            pl.BlockSpec((1, E), fixed),
            pl.BlockSpec((1, E), fixed),
            pl.BlockSpec((D2, EH + E), fixed),
            pl.BlockSpec((1, EH), fixed),
            pl.BlockSpec((EH, E), fixed),
            pl.BlockSpec((1, E), fixed),
            pl.BlockSpec((1, E), fixed),
            pl.BlockSpec((1, 1), fixed),
        ],
        out_specs=pl.BlockSpec((1, block_n), col),
        out_shape=jax.ShapeDtypeStruct((1, B), jnp.float32),
        scratch_shapes=[
            pltpu.VMEM((block_n, EH), jnp.float32),
            pltpu.VMEM((block_n, EH), jnp.float32),
        ],
    )(x1, x2, noise1, noise2,
      w1c, b1f, u1, c1, gb1,
      w2c, b2f, u2, c2, gb2, cb)
    return out.reshape(B, 1)


def kernel(x1, x2, noise1, noise2, E1_W1, E1_b1, E1_W2, E1_b2,
           E2_W1, E2_b1, E2_W2, E2_b2, G1_W, G1_b, G2_W, G2_b, C_W, C_b):
    E, D1, EH = E1_W1.shape
    ED = E1_W2.shape[2]
    D2 = E2_W1.shape[1]

    w1c = jnp.concatenate(
        [jnp.transpose(E1_W1, (1, 0, 2)).reshape(D1, E * EH), G1_W], axis=1)
    b1f = E1_b1.reshape(1, E * EH)
    w2c = jnp.concatenate(
        [jnp.transpose(E2_W1, (1, 0, 2)).reshape(D2, E * EH), G2_W], axis=1)
    b2f = E2_b1.reshape(1, E * EH)
    cw1 = C_W[:ED, 0]
    cw2 = C_W[ED:, 0]
    u1v = jnp.einsum('ehf,f->eh', E1_W2, cw1)
    u2v = jnp.einsum('ehf,f->eh', E2_W2, cw2)
    eye = jnp.eye(E, dtype=jnp.float32)
    u1 = (u1v[:, :, None] * eye[:, None, :]).reshape(E * EH, E)
    u2 = (u2v[:, :, None] * eye[:, None, :]).reshape(E * EH, E)
    c1 = (E1_b2 @ cw1).reshape(1, E)
    c2 = (E2_b2 @ cw2).reshape(1, E)
    gb1 = G1_b.reshape(1, E)
    gb2 = G2_b.reshape(1, E)
    cb = C_b.reshape(1, 1)

    return _moe_fused(x1, x2, noise1, noise2,
                      w1c, b1f, u1, c1, gb1,
                      w2c, b2f, u2, c2, gb2, cb)
